# baseline (speedup 1.0000x reference)
import functools

import jax
import jax.numpy as jnp
from jax import lax
from jax.experimental import pallas as pl
from jax.experimental.pallas import tpu as pltpu
from jax.experimental.pallas import tpu_sc as plsc

NC, NS, L = 2, 16, 16
NW = NC * NS
T = 80
G = (3 * T + 127) // 128
RPG = 3 * T // G
EPS = 1e-8


def _make_sc_kernel(M, C, interpret=False):
    assert M % T == 0 and C % L == 0 and (3 * T) % G == 0 and RPG <= 128
    n_chunks = M // T
    spw = -(-n_chunks // NW)
    spw += spw % 2
    mesh = plsc.VectorSubcoreMesh(
        core_axis_name="c", subcore_axis_name="s",
        num_cores=NC, num_subcores=NS)

    @functools.partial(
        pl.kernel,
        out_type=jax.ShapeDtypeStruct((M, C), jnp.float32),
        mesh=mesh,
        interpret=interpret,
        scratch_types=[
            [pltpu.VMEM((3 * T,), jnp.int32) for _ in range(2)],
            [pltpu.VMEM((3, T), jnp.float32) for _ in range(2)],
            [pltpu.VMEM((3, T), jnp.float32) for _ in range(2)],
            [pltpu.VMEM((3 * T, C), jnp.float32) for _ in range(2)],
            [pltpu.VMEM((T, C), jnp.float32) for _ in range(2)],
            [pltpu.SemaphoreType.DMA for _ in range(2)],
            [pltpu.SemaphoreType.DMA for _ in range(2)],
            [pltpu.SemaphoreType.DMA for _ in range(2)],
        ],
    )
    def k(x_h, ic_h, dc_h, out_h,
          iv, dv, wv, rr, ov, sem_idx, sem_g, sem_out):
        wid = lax.axis_index("s") * NC + lax.axis_index("c")
        slot0 = wid * spw
        last = n_chunks - 1

        def chunk(slot):
            return jnp.minimum(slot, last)

        def fire_idx(slot, b):
            c = chunk(slot)
            pltpu.async_copy(ic_h.at[c], iv[b], sem_idx[b])
            pltpu.async_copy(dc_h.at[c], dv[b], sem_idx[b])

        def drain_idx(b):
            pltpu.make_async_copy(ic_h.at[0], iv[b], sem_idx[b]).wait()
            pltpu.make_async_copy(dc_h.at[0], dv[b], sem_idx[b]).wait()

        def fire_gather(b):
            for g in range(G):
                pltpu.async_copy(
                    x_h.at[iv[b].at[pl.ds(g * RPG, RPG)]],
                    rr[b].at[pl.ds(g * RPG, RPG)], sem_g[b])

        def drain_gather(b):
            for g in range(G):
                pltpu.make_async_copy(
                    x_h.at[pl.ds(0, RPG)],
                    rr[b].at[pl.ds(g * RPG, RPG)], sem_g[b]).wait()

        def weights(b):
            for j in range(T // L):
                sl = pl.ds(j * L, L)
                q0 = 1.0 / (dv[b][0, sl] + EPS)
                q1 = 1.0 / (dv[b][1, sl] + EPS)
                q2 = 1.0 / (dv[b][2, sl] + EPS)
                nrm = q0 + q1 + q2
                wv[b][0, sl] = q0 / nrm
                wv[b][1, sl] = q1 / nrm
                wv[b][2, sl] = q2 / nrm

        def combine(b):
            @plsc.parallel_loop(0, T // L, unroll=2)
            def grp(g):
                w0g = wv[b][0, pl.ds(g * L, L)]
                w1g = wv[b][1, pl.ds(g * L, L)]
                w2g = wv[b][2, pl.ds(g * L, L)]
                for j in range(L):
                    t = g * L + j
                    b0 = jnp.full((L,), w0g[j], jnp.float32)
                    b1 = jnp.full((L,), w1g[j], jnp.float32)
                    b2 = jnp.full((L,), w2g[j], jnp.float32)
                    for kk in range(C // L):
                        s = pl.ds(kk * L, L)
                        ov[b][t, s] = (b0 * rr[b][t, s]
                                       + b1 * rr[b][T + t, s]
                                       + b2 * rr[b][2 * T + t, s])

        def fire_out(slot, b):
            pltpu.async_copy(ov[b], out_h.at[pl.ds(chunk(slot) * T, T)],
                             sem_out[b])

        def drain_out(b):
            pltpu.make_async_copy(ov[b], out_h.at[pl.ds(0, T)],
                                  sem_out[b]).wait()

        def process(slot, b, first):
            weights(b)
            drain_gather(b)
            fire_idx(slot + 2, b)
            if not first:
                drain_out(b)
            combine(b)
            fire_out(slot, b)
            drain_idx(b)
            fire_gather(b)

        fire_idx(slot0, 0)
        fire_idx(slot0 + 1, 1)
        drain_idx(0)
        fire_gather(0)
        drain_idx(1)
        fire_gather(1)
        process(slot0, 0, True)
        process(slot0 + 1, 1, True)

        def pair(i2, carry):
            s = slot0 + i2 * 2
            process(s, 0, False)
            process(s + 1, 1, False)
            return carry

        lax.fori_loop(1, spw // 2, pair, 0)

        for b in range(2):
            drain_gather(b)
            drain_out(b)

    return k


def kernel(x, upsample_inds, upsample_dists):
    M = upsample_inds.shape[0]
    N, C = x.shape
    ic = (upsample_inds.astype(jnp.int32)
          .reshape(M // T, T, 3).transpose(0, 2, 1).reshape(M // T, 3 * T))
    dc = (upsample_dists.astype(jnp.float32)
          .reshape(M // T, T, 3).transpose(0, 2, 1))
    sc = _make_sc_kernel(M, C)
    return sc(x, ic, dc)


# speedup vs baseline: 2.0723x; 2.0723x over previous
import functools

import jax
import jax.numpy as jnp
from jax import lax
from jax.experimental import pallas as pl
from jax.experimental.pallas import tpu as pltpu
from jax.experimental.pallas import tpu_sc as plsc

NC, NS, L = 2, 16, 16
NW = NC * NS
T = 80
G = 3
RPG = 3 * T // G
EPS = 1e-8


def _make_sc_kernel(M, C, interpret=False):
    assert M % T == 0 and C % L == 0 and (3 * T) % G == 0 and RPG <= 128
    n_chunks = M // T
    spw = -(-n_chunks // NW)
    spw += spw % 2
    mesh = plsc.VectorSubcoreMesh(
        core_axis_name="c", subcore_axis_name="s",
        num_cores=NC, num_subcores=NS)

    @functools.partial(
        pl.kernel,
        out_type=jax.ShapeDtypeStruct((M, C), jnp.float32),
        mesh=mesh,
        interpret=interpret,
        scratch_types=[
            [pltpu.VMEM((3 * T,), jnp.int32) for _ in range(2)],
            [pltpu.VMEM((3, T), jnp.float32) for _ in range(2)],
            [pltpu.VMEM((3, T), jnp.float32) for _ in range(2)],
            [pltpu.VMEM((3 * T, C), jnp.float32) for _ in range(2)],
            [pltpu.VMEM((T, C), jnp.float32) for _ in range(2)],
            [pltpu.SemaphoreType.DMA for _ in range(2)],
            [pltpu.SemaphoreType.DMA for _ in range(2)],
            [pltpu.SemaphoreType.DMA for _ in range(2)],
        ],
    )
    def k(x_h, ic_h, dc_h, out_h,
          iv, dv, wv, rr, ov, sem_idx, sem_g, sem_out):
        wid = lax.axis_index("s") * NC + lax.axis_index("c")
        slot0 = wid * spw
        last = n_chunks - 1

        def chunk(slot):
            return jnp.minimum(slot, last)

        def fire_idx(slot, b):
            c = chunk(slot)
            pltpu.async_copy(ic_h.at[c], iv[b], sem_idx[b])
            pltpu.async_copy(dc_h.at[c], dv[b], sem_idx[b])

        def drain_idx(b):
            pltpu.make_async_copy(ic_h.at[0], iv[b], sem_idx[b]).wait()
            pltpu.make_async_copy(dc_h.at[0], dv[b], sem_idx[b]).wait()

        def fire_gather(b):
            for g in range(G):
                pltpu.async_copy(
                    x_h.at[iv[b].at[pl.ds(g * RPG, RPG)]],
                    rr[b].at[pl.ds(g * RPG, RPG)], sem_g[b])

        def drain_gather(b):
            for g in range(G):
                pltpu.make_async_copy(
                    x_h.at[pl.ds(0, RPG)],
                    rr[b].at[pl.ds(g * RPG, RPG)], sem_g[b]).wait()

        def weights(b):
            for j in range(T // L):
                sl = pl.ds(j * L, L)
                q0 = 1.0 / (dv[b][0, sl] + EPS)
                q1 = 1.0 / (dv[b][1, sl] + EPS)
                q2 = 1.0 / (dv[b][2, sl] + EPS)
                nrm = q0 + q1 + q2
                wv[b][0, sl] = q0 / nrm
                wv[b][1, sl] = q1 / nrm
                wv[b][2, sl] = q2 / nrm

        def combine(b):
            def grp(g, c2):
                w0g = wv[b][0, pl.ds(g * L, L)]
                w1g = wv[b][1, pl.ds(g * L, L)]
                w2g = wv[b][2, pl.ds(g * L, L)]
                for j in range(L):
                    t = g * L + j
                    b0 = jnp.full((L,), w0g[j], jnp.float32)
                    b1 = jnp.full((L,), w1g[j], jnp.float32)
                    b2 = jnp.full((L,), w2g[j], jnp.float32)
                    for kk in range(C // L):
                        s = pl.ds(kk * L, L)
                        ov[b][t, s] = (b0 * rr[b][t, s]
                                       + b1 * rr[b][T + t, s]
                                       + b2 * rr[b][2 * T + t, s])
                return c2
            lax.fori_loop(0, T // L, grp, 0)

        def fire_out(slot, b):
            pltpu.async_copy(ov[b], out_h.at[pl.ds(chunk(slot) * T, T)],
                             sem_out[b])

        def drain_out(b):
            pltpu.make_async_copy(ov[b], out_h.at[pl.ds(0, T)],
                                  sem_out[b]).wait()

        def process(slot, b, first):
            weights(b)
            drain_gather(b)
            fire_idx(slot + 2, b)
            if not first:
                drain_out(b)
            combine(b)
            fire_out(slot, b)
            drain_idx(b)
            fire_gather(b)

        fire_idx(slot0, 0)
        fire_idx(slot0 + 1, 1)
        drain_idx(0)
        fire_gather(0)
        drain_idx(1)
        fire_gather(1)
        process(slot0, 0, True)
        process(slot0 + 1, 1, True)

        def pair(i2, carry):
            s = slot0 + i2 * 2
            process(s, 0, False)
            process(s + 1, 1, False)
            return carry

        lax.fori_loop(1, spw // 2, pair, 0)

        for b in range(2):
            drain_gather(b)
            drain_out(b)

    return k


def kernel(x, upsample_inds, upsample_dists):
    M = upsample_inds.shape[0]
    N, C = x.shape
    ic = (upsample_inds.astype(jnp.int32)
          .reshape(M // T, T, 3).transpose(0, 2, 1).reshape(M // T, 3 * T))
    dc = (upsample_dists.astype(jnp.float32)
          .reshape(M // T, T, 3).transpose(0, 2, 1))
    sc = _make_sc_kernel(M, C)
    return sc(x, ic, dc)


# speedup vs baseline: 2.2600x; 1.0906x over previous
import functools

import jax
import jax.numpy as jnp
from jax import lax
from jax.experimental import pallas as pl
from jax.experimental.pallas import tpu as pltpu
from jax.experimental.pallas import tpu_sc as plsc

NC, NS, L = 2, 16, 16
NW = NC * NS
T = 80
G = 3
RPG = 3 * T // G
EPS = 1e-8


def _make_sc_kernel(M, C, interpret=False):
    assert M % T == 0 and C % L == 0 and (3 * T) % G == 0 and RPG <= 128
    n_chunks = M // T
    spw = -(-n_chunks // NW)
    spw += spw % 2
    mesh = plsc.VectorSubcoreMesh(
        core_axis_name="c", subcore_axis_name="s",
        num_cores=NC, num_subcores=NS)

    @functools.partial(
        pl.kernel,
        out_type=jax.ShapeDtypeStruct((M, C), jnp.float32),
        mesh=mesh,
        interpret=interpret,
        scratch_types=[
            [pltpu.VMEM((3 * T,), jnp.int32) for _ in range(2)],
            [pltpu.VMEM((3, T), jnp.float32) for _ in range(2)],
            [pltpu.VMEM((3, T), jnp.float32) for _ in range(2)],
            [pltpu.VMEM((3 * T, C), jnp.float32) for _ in range(2)],
            [pltpu.VMEM((T, C), jnp.float32) for _ in range(2)],
            [pltpu.SemaphoreType.DMA for _ in range(2)],
            [pltpu.SemaphoreType.DMA for _ in range(2)],
            [pltpu.SemaphoreType.DMA for _ in range(2)],
        ],
    )
    def k(x_h, ic_h, dc_h, out_h,
          iv, dv, wv, rr, ov, sem_idx, sem_g, sem_out):
        wid = lax.axis_index("s") * NC + lax.axis_index("c")
        slot0 = wid * spw
        last = n_chunks - 1

        def chunk(slot):
            return jnp.minimum(slot, last)

        def fire_idx(slot, b):
            c = chunk(slot)
            pltpu.async_copy(ic_h.at[c], iv[b], sem_idx[b])
            pltpu.async_copy(dc_h.at[c], dv[b], sem_idx[b])

        def drain_idx(b):
            pltpu.make_async_copy(ic_h.at[0], iv[b], sem_idx[b]).wait()
            pltpu.make_async_copy(dc_h.at[0], dv[b], sem_idx[b]).wait()

        def fire_gather(b):
            for g in range(G):
                pltpu.async_copy(
                    x_h.at[iv[b].at[pl.ds(g * RPG, RPG)]],
                    rr[b].at[pl.ds(g * RPG, RPG)], sem_g[b])

        def drain_gather(b):
            for g in range(G):
                pltpu.make_async_copy(
                    x_h.at[pl.ds(0, RPG)],
                    rr[b].at[pl.ds(g * RPG, RPG)], sem_g[b]).wait()

        def weights(b):
            for j in range(T // L):
                sl = pl.ds(j * L, L)
                e0 = dv[b][0, sl] + EPS
                e1 = dv[b][1, sl] + EPS
                e2 = dv[b][2, sl] + EPS
                p01 = e0 * e1
                p12 = e1 * e2
                p02 = e0 * e2
                inv = 1.0 / (p01 + p12 + p02)
                wv[b][0, sl] = p12 * inv
                wv[b][1, sl] = p02 * inv
                wv[b][2, sl] = p01 * inv

        def combine(b):
            def grp(g, c2):
                w0g = wv[b][0, pl.ds(g * L, L)]
                w1g = wv[b][1, pl.ds(g * L, L)]
                w2g = wv[b][2, pl.ds(g * L, L)]
                for j in range(L):
                    t = g * L + j
                    b0 = jnp.full((L,), w0g[j], jnp.float32)
                    b1 = jnp.full((L,), w1g[j], jnp.float32)
                    b2 = jnp.full((L,), w2g[j], jnp.float32)
                    for kk in range(C // L):
                        s = pl.ds(kk * L, L)
                        ov[b][t, s] = (b0 * rr[b][t, s]
                                       + b1 * rr[b][T + t, s]
                                       + b2 * rr[b][2 * T + t, s])
                return c2
            lax.fori_loop(0, T // L, grp, 0)

        def fire_out(slot, b):
            pltpu.async_copy(ov[b], out_h.at[pl.ds(chunk(slot) * T, T)],
                             sem_out[b])

        def drain_out(b):
            pltpu.make_async_copy(ov[b], out_h.at[pl.ds(0, T)],
                                  sem_out[b]).wait()

        def process(slot, b, first):
            weights(b)
            drain_gather(b)
            fire_idx(slot + 2, b)
            if not first:
                drain_out(b)
            combine(b)
            fire_out(slot, b)
            drain_idx(b)
            fire_gather(b)

        fire_idx(slot0, 0)
        fire_idx(slot0 + 1, 1)
        drain_idx(0)
        fire_gather(0)
        drain_idx(1)
        fire_gather(1)
        process(slot0, 0, True)
        process(slot0 + 1, 1, True)

        def pair(i2, carry):
            s = slot0 + i2 * 2
            process(s, 0, False)
            process(s + 1, 1, False)
            return carry

        lax.fori_loop(1, spw // 2, pair, 0)

        for b in range(2):
            drain_gather(b)
            drain_out(b)

    return k


def kernel(x, upsample_inds, upsample_dists):
    M = upsample_inds.shape[0]
    N, C = x.shape
    ic = (upsample_inds.astype(jnp.int32)
          .reshape(M // T, T, 3).transpose(0, 2, 1).reshape(M // T, 3 * T))
    dc = (upsample_dists.astype(jnp.float32)
          .reshape(M // T, T, 3).transpose(0, 2, 1))
    sc = _make_sc_kernel(M, C)
    return sc(x, ic, dc)
